# baseline (speedup 1.0000x reference)
import functools

import jax
import jax.numpy as jnp
from jax import lax
from jax.experimental import pallas as pl
from jax.experimental.pallas import tpu as pltpu
from jax.experimental.pallas import tpu_sc as plsc

NC = 2
NS = 16
NW = NC * NS
LANES = 16


def _sc_pool(tok_mat, emb_table):
    B, L = tok_mat.shape
    V, EMB = emb_table.shape
    bags_per_w = B // NW
    n_vec = EMB // LANES
    chunks = []
    off = 0
    while off < L:
        sz = min(128, L - off)
        chunks.append((off, sz))
        off += sz

    mesh = plsc.VectorSubcoreMesh(core_axis_name="c", subcore_axis_name="s")

    @functools.partial(
        pl.kernel,
        out_type=jax.ShapeDtypeStruct((B, EMB), jnp.float32),
        mesh=mesh,
        scratch_types=[
            pltpu.VMEM((L,), jnp.int32),
            pltpu.VMEM((L, EMB), jnp.float32),
            pltpu.VMEM((EMB,), jnp.float32),
            pltpu.SemaphoreType.DMA,
        ],
        compiler_params=pltpu.CompilerParams(use_tc_tiling_on_sc=False),
    )
    def pool(tok_hbm, table_hbm, out_hbm, idx_v, rows_v, acc_v, sem):
        wid = lax.axis_index("s") * NC + lax.axis_index("c")
        base = wid * bags_per_w

        def bag_body(i, carry):
            b = base + i
            pltpu.sync_copy(tok_hbm.at[b], idx_v)
            cps = []
            for off, sz in chunks:
                cps.append(
                    pltpu.async_copy(
                        table_hbm.at[idx_v.at[pl.ds(off, sz)]],
                        rows_v.at[pl.ds(off, sz)],
                        sem,
                    )
                )
            for cp in cps:
                cp.wait()

            def row_body(j, accs):
                return tuple(
                    accs[k] + rows_v[j, pl.ds(k * LANES, LANES)]
                    for k in range(n_vec)
                )

            zeros = tuple(jnp.zeros((LANES,), jnp.float32) for _ in range(n_vec))
            accs = lax.fori_loop(0, L, row_body, zeros)
            for k in range(n_vec):
                acc_v[pl.ds(k * LANES, LANES)] = accs[k]
            pltpu.sync_copy(acc_v, out_hbm.at[b])
            return carry

        lax.fori_loop(0, bags_per_w, bag_body, 0)

    return pool(tok_mat, emb_table)


def _tc_heads(pooled_sums, tok_mat, mask, symb_feats, row0,
              W1a, W1b, b1, W2, b2, Wsv, bsv, Wcv, bcv):
    B, L = tok_mat.shape
    EMB = pooled_sums.shape[1]
    S = symb_feats.shape[1]
    H = W1a.shape[1]
    NCLS = W2.shape[1]
    RB = 512
    grid = (B // RB,)

    def body(sums_ref, tok_ref, mask_ref, symb_ref, row0_ref,
             W1a_ref, W1b_ref, b1_ref, W2_ref, b2_ref,
             Wsv_ref, bsv_ref, Wcv_ref, bcv_ref,
             cls_ref, sv_ref, cv_ref):
        tok = tok_ref[...]
        cnt0 = jnp.sum((tok == 0).astype(jnp.float32), axis=1, keepdims=True)
        msum = jnp.sum(mask_ref[...], axis=1, keepdims=True)
        pooled = (sums_ref[...] - cnt0 * row0_ref[...]) / msum
        h = jnp.dot(pooled, W1a_ref[...], preferred_element_type=jnp.float32)
        h = h + jnp.dot(symb_ref[...], W1b_ref[...],
                        preferred_element_type=jnp.float32)
        h = jnp.maximum(h + b1_ref[...], 0.0)
        cls_ref[...] = (
            jnp.dot(h, W2_ref[...], preferred_element_type=jnp.float32)
            + b2_ref[...]
        )
        sv_ref[...] = (
            jnp.dot(pooled, Wsv_ref[...], preferred_element_type=jnp.float32)
            + bsv_ref[...]
        )
        cv_ref[...] = (
            jnp.dot(pooled, Wcv_ref[...], preferred_element_type=jnp.float32)
            + bcv_ref[...]
        )

    row_spec = pl.BlockSpec((RB, None), lambda i: (i, 0))
    full = pl.BlockSpec(lambda i: (0, 0))
    cls, sv, cv = pl.pallas_call(
        body,
        grid=grid,
        in_specs=[
            pl.BlockSpec((RB, EMB), lambda i: (i, 0)),
            pl.BlockSpec((RB, L), lambda i: (i, 0)),
            pl.BlockSpec((RB, L), lambda i: (i, 0)),
            pl.BlockSpec((RB, S), lambda i: (i, 0)),
            pl.BlockSpec((1, EMB), lambda i: (0, 0)),
            pl.BlockSpec((EMB, H), lambda i: (0, 0)),
            pl.BlockSpec((S, H), lambda i: (0, 0)),
            pl.BlockSpec((1, H), lambda i: (0, 0)),
            pl.BlockSpec((H, NCLS), lambda i: (0, 0)),
            pl.BlockSpec((1, NCLS), lambda i: (0, 0)),
            pl.BlockSpec((EMB, 1), lambda i: (0, 0)),
            pl.BlockSpec((1, 1), lambda i: (0, 0)),
            pl.BlockSpec((EMB, 1), lambda i: (0, 0)),
            pl.BlockSpec((1, 1), lambda i: (0, 0)),
        ],
        out_specs=[
            pl.BlockSpec((RB, NCLS), lambda i: (i, 0)),
            pl.BlockSpec((RB, 1), lambda i: (i, 0)),
            pl.BlockSpec((RB, 1), lambda i: (i, 0)),
        ],
        out_shape=[
            jax.ShapeDtypeStruct((B, NCLS), jnp.float32),
            jax.ShapeDtypeStruct((B, 1), jnp.float32),
            jax.ShapeDtypeStruct((B, 1), jnp.float32),
        ],
    )(pooled_sums, tok_mat, mask, symb_feats, row0,
      W1a, W1b, b1.reshape(1, H), W2, b2.reshape(1, NCLS),
      Wsv, bsv.reshape(1, 1), Wcv, bcv.reshape(1, 1))
    return cls, sv.reshape(B), cv.reshape(B)


def kernel(tok_mat, mask, symb_feats, emb_table, W1, b1, W2, b2,
           Wsv, bsv, Wcv, bcv):
    EMB = emb_table.shape[1]
    pooled_sums = _sc_pool(tok_mat, emb_table)
    row0 = emb_table[0:1, :]
    W1a = W1[:EMB, :]
    W1b = W1[EMB:, :]
    return _tc_heads(pooled_sums, tok_mat, mask, symb_feats, row0,
                     W1a, W1b, b1, W2, b2, Wsv, bsv, Wcv, bcv)


# speedup vs baseline: 1.0520x; 1.0520x over previous
import functools

import jax
import jax.numpy as jnp
from jax import lax
from jax.experimental import pallas as pl
from jax.experimental.pallas import tpu as pltpu
from jax.experimental.pallas import tpu_sc as plsc

NC = 2
NS = 16
NW = NC * NS
LANES = 16


def _sc_pool(tok_mat, emb_table):
    B, L = tok_mat.shape
    V, EMB = emb_table.shape
    bags_per_w = B // NW
    n_vec = EMB // LANES
    chunks = []
    off = 0
    while off < L:
        sz = min(128, L - off)
        chunks.append((off, sz))
        off += sz

    mesh = plsc.VectorSubcoreMesh(core_axis_name="c", subcore_axis_name="s")

    @functools.partial(
        pl.kernel,
        out_type=jax.ShapeDtypeStruct((B, EMB), jnp.float32),
        mesh=mesh,
        scratch_types=[
            pltpu.VMEM((L,), jnp.int32),
            pltpu.VMEM((L, EMB), jnp.float32),
            pltpu.VMEM((EMB,), jnp.float32),
            pltpu.SemaphoreType.DMA,
        ],
        compiler_params=pltpu.CompilerParams(use_tc_tiling_on_sc=False),
    )
    def pool(tok_hbm, table_hbm, out_hbm, idx_v, rows_v, acc_v, sem):
        wid = lax.axis_index("s") * NC + lax.axis_index("c")
        base = wid * bags_per_w

        def bag_body(i, carry):
            b = base + i
            pltpu.sync_copy(tok_hbm.at[b], idx_v)
            cps = []
            for off, sz in chunks:
                cps.append(
                    pltpu.async_copy(
                        table_hbm.at[idx_v.at[pl.ds(off, sz)]],
                        rows_v.at[pl.ds(off, sz)],
                        sem,
                    )
                )
            for cp in cps:
                cp.wait()

            def row_body(j, accs):
                return tuple(
                    accs[k] + rows_v[j, pl.ds(k * LANES, LANES)]
                    for k in range(n_vec)
                )

            zeros = tuple(jnp.zeros((LANES,), jnp.float32) for _ in range(n_vec))
            accs = lax.fori_loop(0, L, row_body, zeros)
            for k in range(n_vec):
                acc_v[pl.ds(k * LANES, LANES)] = accs[k]
            pltpu.sync_copy(acc_v, out_hbm.at[b])
            return carry

        lax.fori_loop(0, bags_per_w, bag_body, 0)

    return pool(tok_mat, emb_table)


def _tc_pack(tableT):
    EMB, V = tableT.shape
    W = 2048
    grid = ((V + W - 1) // W,)

    def body(t_ref, o_ref):
        t = t_ref[...]
        tt = jnp.transpose(t)
        t3 = tt.reshape(W // 2, 2, EMB)
        o_ref[:, 0:EMB] = t3[:, 0, :]
        o_ref[:, EMB:2 * EMB] = t3[:, 1, :]

    return pl.pallas_call(
        body,
        grid=grid,
        in_specs=[pl.BlockSpec((EMB, W), lambda i: (0, i))],
        out_specs=pl.BlockSpec((W // 2, 2 * EMB), lambda i: (i, 0)),
        out_shape=jax.ShapeDtypeStruct((V // 2, 2 * EMB), jnp.float32),
    )(tableT)


def _tc_heads(pooled_sums, tok_mat, mask, symb_feats, row0,
              W1a, W1b, b1, W2, b2, Wsv, bsv, Wcv, bcv):
    B, L = tok_mat.shape
    EMB = pooled_sums.shape[1]
    S = symb_feats.shape[1]
    H = W1a.shape[1]
    NCLS = W2.shape[1]
    RB = 512
    grid = (B // RB,)

    def body(sums_ref, tok_ref, mask_ref, symb_ref, row0_ref,
             W1a_ref, W1b_ref, b1_ref, W2_ref, b2_ref,
             Wsv_ref, bsv_ref, Wcv_ref, bcv_ref,
             cls_ref, sv_ref, cv_ref):
        tok = tok_ref[...]
        cnt0 = jnp.sum((tok == 0).astype(jnp.float32), axis=1, keepdims=True)
        msum = jnp.sum(mask_ref[...], axis=1, keepdims=True)
        pooled = (sums_ref[...] - cnt0 * row0_ref[...]) / msum
        h = jnp.dot(pooled, W1a_ref[...], preferred_element_type=jnp.float32)
        h = h + jnp.dot(symb_ref[...], W1b_ref[...],
                        preferred_element_type=jnp.float32)
        h = jnp.maximum(h + b1_ref[...], 0.0)
        cls_ref[...] = (
            jnp.dot(h, W2_ref[...], preferred_element_type=jnp.float32)
            + b2_ref[...]
        )
        sv_ref[...] = (
            jnp.dot(pooled, Wsv_ref[...], preferred_element_type=jnp.float32)
            + bsv_ref[...]
        )
        cv_ref[...] = (
            jnp.dot(pooled, Wcv_ref[...], preferred_element_type=jnp.float32)
            + bcv_ref[...]
        )

    row_spec = pl.BlockSpec((RB, None), lambda i: (i, 0))
    full = pl.BlockSpec(lambda i: (0, 0))
    cls, sv, cv = pl.pallas_call(
        body,
        grid=grid,
        in_specs=[
            pl.BlockSpec((RB, EMB), lambda i: (i, 0)),
            pl.BlockSpec((RB, L), lambda i: (i, 0)),
            pl.BlockSpec((RB, L), lambda i: (i, 0)),
            pl.BlockSpec((RB, S), lambda i: (i, 0)),
            pl.BlockSpec((1, EMB), lambda i: (0, 0)),
            pl.BlockSpec((EMB, H), lambda i: (0, 0)),
            pl.BlockSpec((S, H), lambda i: (0, 0)),
            pl.BlockSpec((1, H), lambda i: (0, 0)),
            pl.BlockSpec((H, NCLS), lambda i: (0, 0)),
            pl.BlockSpec((1, NCLS), lambda i: (0, 0)),
            pl.BlockSpec((EMB, 1), lambda i: (0, 0)),
            pl.BlockSpec((1, 1), lambda i: (0, 0)),
            pl.BlockSpec((EMB, 1), lambda i: (0, 0)),
            pl.BlockSpec((1, 1), lambda i: (0, 0)),
        ],
        out_specs=[
            pl.BlockSpec((RB, NCLS), lambda i: (i, 0)),
            pl.BlockSpec((RB, 1), lambda i: (i, 0)),
            pl.BlockSpec((RB, 1), lambda i: (i, 0)),
        ],
        out_shape=[
            jax.ShapeDtypeStruct((B, NCLS), jnp.float32),
            jax.ShapeDtypeStruct((B, 1), jnp.float32),
            jax.ShapeDtypeStruct((B, 1), jnp.float32),
        ],
    )(pooled_sums, tok_mat, mask, symb_feats, row0,
      W1a, W1b, b1.reshape(1, H), W2, b2.reshape(1, NCLS),
      Wsv, bsv.reshape(1, 1), Wcv, bcv.reshape(1, 1))
    return cls, sv.reshape(B), cv.reshape(B)


def kernel(tok_mat, mask, symb_feats, emb_table, W1, b1, W2, b2,
           Wsv, bsv, Wcv, bcv):
    V, EMB = emb_table.shape
    packed = _tc_pack(emb_table.T)
    table_lin = packed.reshape(V, EMB)
    pooled_sums = _sc_pool(tok_mat, table_lin)
    row0 = emb_table[0:1, :]
    W1a = W1[:EMB, :]
    W1b = W1[EMB:, :]
    return _tc_heads(pooled_sums, tok_mat, mask, symb_feats, row0,
                     W1a, W1b, b1, W2, b2, Wsv, bsv, Wcv, bcv)


# speedup vs baseline: 1.3878x; 1.3192x over previous
import functools

import jax
import jax.numpy as jnp
from jax import lax
from jax.experimental import pallas as pl
from jax.experimental.pallas import tpu as pltpu
from jax.experimental.pallas import tpu_sc as plsc

NC = 2
NS = 16
NW = NC * NS
LANES = 16


def _sc_pool(tok_mat, table_lin, vh):
    B, L = tok_mat.shape
    EMB = table_lin.shape[1]
    bags_per_w = B // NW
    n_vec = EMB // LANES
    lpad = ((L + LANES - 1) // LANES) * LANES
    chunks = []
    off = 0
    while off < L:
        sz = min(128, L - off)
        chunks.append((off, sz))
        off += sz

    mesh = plsc.VectorSubcoreMesh(core_axis_name="c", subcore_axis_name="s")

    @functools.partial(
        pl.kernel,
        out_type=jax.ShapeDtypeStruct((B, EMB), jnp.float32),
        mesh=mesh,
        scratch_types=[
            pltpu.VMEM((lpad,), jnp.int32),
            pltpu.VMEM((lpad,), jnp.int32),
            pltpu.VMEM((L, EMB), jnp.float32),
            pltpu.VMEM((EMB,), jnp.float32),
            pltpu.SemaphoreType.DMA,
        ],
        compiler_params=pltpu.CompilerParams(use_tc_tiling_on_sc=False),
    )
    def pool(tok_hbm, table_hbm, out_hbm, idx_v, jdx_v, rows_v, acc_v, sem):
        wid = lax.axis_index("s") * NC + lax.axis_index("c")
        base = wid * bags_per_w

        def bag_body(i, carry):
            b = base + i
            pltpu.sync_copy(tok_hbm.at[b], idx_v.at[pl.ds(0, L)])
            for c in range(lpad // LANES):
                v = idx_v[pl.ds(c * LANES, LANES)]
                dbl = v + v
                jdx_v[pl.ds(c * LANES, LANES)] = jnp.where(
                    v >= vh, dbl - (2 * vh - 1), dbl)
            cps = []
            for off, sz in chunks:
                cps.append(
                    pltpu.async_copy(
                        table_hbm.at[jdx_v.at[pl.ds(off, sz)]],
                        rows_v.at[pl.ds(off, sz)],
                        sem,
                    )
                )
            for cp in cps:
                cp.wait()

            def row_body(j, accs):
                return tuple(
                    accs[k] + rows_v[j, pl.ds(k * LANES, LANES)]
                    for k in range(n_vec)
                )

            zeros = tuple(jnp.zeros((LANES,), jnp.float32) for _ in range(n_vec))
            accs = lax.fori_loop(0, L, row_body, zeros)
            for k in range(n_vec):
                acc_v[pl.ds(k * LANES, LANES)] = accs[k]
            pltpu.sync_copy(acc_v, out_hbm.at[b])
            return carry

        lax.fori_loop(0, bags_per_w, bag_body, 0)

    return pool(tok_mat, table_lin)


def _tc_pack(tableT):
    EMB, V = tableT.shape
    W = 2048
    nb = (V // 2 + W - 1) // W
    vh = nb * W

    def body(ta_ref, tb_ref, o_ref):
        o_ref[:, 0:EMB] = jnp.transpose(ta_ref[...])
        o_ref[:, EMB:2 * EMB] = jnp.transpose(tb_ref[...])

    packed = pl.pallas_call(
        body,
        grid=(nb,),
        in_specs=[pl.BlockSpec((EMB, W), lambda i: (0, i)),
                  pl.BlockSpec((EMB, W),
                               lambda i: (0, jnp.minimum(i + nb,
                                                         V // W)))],
        out_specs=pl.BlockSpec((W, 2 * EMB), lambda i: (i, 0)),
        out_shape=jax.ShapeDtypeStruct((vh, 2 * EMB), jnp.float32),
    )(tableT, tableT)
    return packed, vh


def _tc_heads(pooled_sums, tok_mat, mask, symb_feats, row0,
              W1a, W1b, b1, W2, b2, Wsv, bsv, Wcv, bcv):
    B, L = tok_mat.shape
    EMB = pooled_sums.shape[1]
    S = symb_feats.shape[1]
    H = W1a.shape[1]
    NCLS = W2.shape[1]
    RB = 512
    grid = (B // RB,)

    def body(sums_ref, tok_ref, mask_ref, symb_ref, row0_ref,
             W1a_ref, W1b_ref, b1_ref, W2_ref, b2_ref,
             Wsv_ref, bsv_ref, Wcv_ref, bcv_ref,
             cls_ref, sv_ref, cv_ref):
        tok = tok_ref[...]
        cnt0 = jnp.sum((tok == 0).astype(jnp.float32), axis=1, keepdims=True)
        msum = jnp.sum(mask_ref[...], axis=1, keepdims=True)
        pooled = (sums_ref[...] - cnt0 * row0_ref[...]) / msum
        h = jnp.dot(pooled, W1a_ref[...], preferred_element_type=jnp.float32)
        h = h + jnp.dot(symb_ref[...], W1b_ref[...],
                        preferred_element_type=jnp.float32)
        h = jnp.maximum(h + b1_ref[...], 0.0)
        cls_ref[...] = (
            jnp.dot(h, W2_ref[...], preferred_element_type=jnp.float32)
            + b2_ref[...]
        )
        sv_ref[...] = (
            jnp.dot(pooled, Wsv_ref[...], preferred_element_type=jnp.float32)
            + bsv_ref[...]
        )
        cv_ref[...] = (
            jnp.dot(pooled, Wcv_ref[...], preferred_element_type=jnp.float32)
            + bcv_ref[...]
        )

    row_spec = pl.BlockSpec((RB, None), lambda i: (i, 0))
    full = pl.BlockSpec(lambda i: (0, 0))
    cls, sv, cv = pl.pallas_call(
        body,
        grid=grid,
        in_specs=[
            pl.BlockSpec((RB, EMB), lambda i: (i, 0)),
            pl.BlockSpec((RB, L), lambda i: (i, 0)),
            pl.BlockSpec((RB, L), lambda i: (i, 0)),
            pl.BlockSpec((RB, S), lambda i: (i, 0)),
            pl.BlockSpec((1, EMB), lambda i: (0, 0)),
            pl.BlockSpec((EMB, H), lambda i: (0, 0)),
            pl.BlockSpec((S, H), lambda i: (0, 0)),
            pl.BlockSpec((1, H), lambda i: (0, 0)),
            pl.BlockSpec((H, NCLS), lambda i: (0, 0)),
            pl.BlockSpec((1, NCLS), lambda i: (0, 0)),
            pl.BlockSpec((EMB, 1), lambda i: (0, 0)),
            pl.BlockSpec((1, 1), lambda i: (0, 0)),
            pl.BlockSpec((EMB, 1), lambda i: (0, 0)),
            pl.BlockSpec((1, 1), lambda i: (0, 0)),
        ],
        out_specs=[
            pl.BlockSpec((RB, NCLS), lambda i: (i, 0)),
            pl.BlockSpec((RB, 1), lambda i: (i, 0)),
            pl.BlockSpec((RB, 1), lambda i: (i, 0)),
        ],
        out_shape=[
            jax.ShapeDtypeStruct((B, NCLS), jnp.float32),
            jax.ShapeDtypeStruct((B, 1), jnp.float32),
            jax.ShapeDtypeStruct((B, 1), jnp.float32),
        ],
    )(pooled_sums, tok_mat, mask, symb_feats, row0,
      W1a, W1b, b1.reshape(1, H), W2, b2.reshape(1, NCLS),
      Wsv, bsv.reshape(1, 1), Wcv, bcv.reshape(1, 1))
    return cls, sv.reshape(B), cv.reshape(B)


def kernel(tok_mat, mask, symb_feats, emb_table, W1, b1, W2, b2,
           Wsv, bsv, Wcv, bcv):
    V, EMB = emb_table.shape
    packed, vh = _tc_pack(emb_table.T)
    table_lin = packed.reshape(2 * vh, EMB)
    pooled_sums = _sc_pool(tok_mat, table_lin, vh)
    row0 = emb_table[0:1, :]
    W1a = W1[:EMB, :]
    W1b = W1[EMB:, :]
    return _tc_heads(pooled_sums, tok_mat, mask, symb_feats, row0,
                     W1a, W1b, b1, W2, b2, Wsv, bsv, Wcv, bcv)


# speedup vs baseline: 1.9110x; 1.3771x over previous
import functools

import jax
import jax.numpy as jnp
from jax import lax
from jax.experimental import pallas as pl
from jax.experimental.pallas import tpu as pltpu
from jax.experimental.pallas import tpu_sc as plsc

NC = 2
NS = 16
NW = NC * NS
LANES = 16


def _sc_pool(tok_flat, table_lin, vh, B, L):
    EMB = table_lin.shape[1]
    bags_per_w = B // NW
    G = 4
    GL = G * L
    n_groups = bags_per_w // G
    n_vec = EMB // LANES
    bag_chunks = [(0, 128), (128, L - 128)]

    mesh = plsc.VectorSubcoreMesh(core_axis_name="c", subcore_axis_name="s")

    @functools.partial(
        pl.kernel,
        out_type=jax.ShapeDtypeStruct((B, EMB), jnp.float32),
        mesh=mesh,
        scratch_types=[
            pltpu.VMEM((GL,), jnp.int32),
            pltpu.VMEM((GL,), jnp.int32),
            pltpu.VMEM((GL,), jnp.int32),
            pltpu.VMEM((GL,), jnp.int32),
            pltpu.VMEM((GL, EMB), jnp.float32),
            pltpu.VMEM((GL, EMB), jnp.float32),
            pltpu.VMEM((G, EMB), jnp.float32),
            pltpu.SemaphoreType.DMA,
            pltpu.SemaphoreType.DMA,
        ],
        compiler_params=pltpu.CompilerParams(use_tc_tiling_on_sc=False),
    )
    def pool(tok_hbm, table_hbm, out_hbm,
             idx0, idx1, jdx0, jdx1, rows0, rows1, acc_v, sem0, sem1):
        wid = lax.axis_index("s") * NC + lax.axis_index("c")
        base = wid * bags_per_w
        idx = (idx0, idx1)
        jdx = (jdx0, jdx1)
        rows = (rows0, rows1)
        sems = (sem0, sem1)

        def load_and_fire(g, pp):
            start = (base + g * G) * L
            pltpu.sync_copy(tok_hbm.at[pl.ds(start, GL)], idx[pp])
            for c in range(GL // LANES):
                v = idx[pp][pl.ds(c * LANES, LANES)]
                dbl = v + v
                jdx[pp][pl.ds(c * LANES, LANES)] = jnp.where(
                    v >= vh, dbl - (2 * vh - 1), dbl)
            for m in range(G):
                for off, sz in bag_chunks:
                    pltpu.async_copy(
                        table_hbm.at[jdx[pp].at[pl.ds(m * L + off, sz)]],
                        rows[pp].at[pl.ds(m * L + off, sz)],
                        sems[pp],
                    )

        def wait_and_acc(g, pp):
            for m in range(G):
                for off, sz in bag_chunks:
                    pltpu.make_async_copy(
                        table_hbm.at[jdx[pp].at[pl.ds(m * L + off, sz)]],
                        rows[pp].at[pl.ds(m * L + off, sz)],
                        sems[pp],
                    ).wait()
            for m in range(G):
                def row_body(j, accs):
                    return tuple(
                        accs[k] + rows[pp][j, pl.ds(k * LANES, LANES)]
                        for k in range(n_vec)
                    )
                zeros = tuple(jnp.zeros((LANES,), jnp.float32)
                              for _ in range(n_vec))
                accs = lax.fori_loop(m * L, (m + 1) * L, row_body, zeros)
                for k in range(n_vec):
                    acc_v[m, pl.ds(k * LANES, LANES)] = accs[k]
            pltpu.sync_copy(acc_v, out_hbm.at[pl.ds(base + g * G, G)])

        load_and_fire(0, 0)

        def body2(k, carry):
            g = 2 * k
            load_and_fire(g + 1, 1)
            wait_and_acc(g, 0)
            load_and_fire(g + 2, 0)
            wait_and_acc(g + 1, 1)
            return carry

        lax.fori_loop(0, n_groups // 2 - 1, body2, 0)
        g_last = n_groups - 2
        load_and_fire(g_last + 1, 1)
        wait_and_acc(g_last, 0)
        wait_and_acc(g_last + 1, 1)

    return pool(tok_flat, table_lin)


def _tc_pack(tableT):
    EMB, V = tableT.shape
    W = 2048
    nb = (V // 2 + W - 1) // W
    vh = nb * W

    def body(ta_ref, tb_ref, o_ref):
        o_ref[:, 0:EMB] = jnp.transpose(ta_ref[...])
        o_ref[:, EMB:2 * EMB] = jnp.transpose(tb_ref[...])

    packed = pl.pallas_call(
        body,
        grid=(nb,),
        in_specs=[pl.BlockSpec((EMB, W), lambda i: (0, i)),
                  pl.BlockSpec((EMB, W),
                               lambda i: (0, jnp.minimum(i + nb,
                                                         V // W)))],
        out_specs=pl.BlockSpec((W, 2 * EMB), lambda i: (i, 0)),
        out_shape=jax.ShapeDtypeStruct((vh, 2 * EMB), jnp.float32),
    )(tableT, tableT)
    return packed, vh


def _tc_heads(pooled_sums, tok_mat, mask, symb_feats, row0,
              W1a, W1b, b1, W2, b2, Wsv, bsv, Wcv, bcv):
    B, L = tok_mat.shape
    EMB = pooled_sums.shape[1]
    S = symb_feats.shape[1]
    H = W1a.shape[1]
    NCLS = W2.shape[1]
    RB = 512
    grid = (B // RB,)

    def body(sums_ref, tok_ref, mask_ref, symb_ref, row0_ref,
             W1a_ref, W1b_ref, b1_ref, W2_ref, b2_ref,
             Wsv_ref, bsv_ref, Wcv_ref, bcv_ref,
             cls_ref, sv_ref, cv_ref):
        tok = tok_ref[...]
        cnt0 = jnp.sum((tok == 0).astype(jnp.float32), axis=1, keepdims=True)
        msum = jnp.sum(mask_ref[...], axis=1, keepdims=True)
        pooled = (sums_ref[...] - cnt0 * row0_ref[...]) / msum
        h = jnp.dot(pooled, W1a_ref[...], preferred_element_type=jnp.float32)
        h = h + jnp.dot(symb_ref[...], W1b_ref[...],
                        preferred_element_type=jnp.float32)
        h = jnp.maximum(h + b1_ref[...], 0.0)
        cls_ref[...] = (
            jnp.dot(h, W2_ref[...], preferred_element_type=jnp.float32)
            + b2_ref[...]
        )
        sv_ref[...] = (
            jnp.dot(pooled, Wsv_ref[...], preferred_element_type=jnp.float32)
            + bsv_ref[...]
        )
        cv_ref[...] = (
            jnp.dot(pooled, Wcv_ref[...], preferred_element_type=jnp.float32)
            + bcv_ref[...]
        )

    row_spec = pl.BlockSpec((RB, None), lambda i: (i, 0))
    full = pl.BlockSpec(lambda i: (0, 0))
    cls, sv, cv = pl.pallas_call(
        body,
        grid=grid,
        in_specs=[
            pl.BlockSpec((RB, EMB), lambda i: (i, 0)),
            pl.BlockSpec((RB, L), lambda i: (i, 0)),
            pl.BlockSpec((RB, L), lambda i: (i, 0)),
            pl.BlockSpec((RB, S), lambda i: (i, 0)),
            pl.BlockSpec((1, EMB), lambda i: (0, 0)),
            pl.BlockSpec((EMB, H), lambda i: (0, 0)),
            pl.BlockSpec((S, H), lambda i: (0, 0)),
            pl.BlockSpec((1, H), lambda i: (0, 0)),
            pl.BlockSpec((H, NCLS), lambda i: (0, 0)),
            pl.BlockSpec((1, NCLS), lambda i: (0, 0)),
            pl.BlockSpec((EMB, 1), lambda i: (0, 0)),
            pl.BlockSpec((1, 1), lambda i: (0, 0)),
            pl.BlockSpec((EMB, 1), lambda i: (0, 0)),
            pl.BlockSpec((1, 1), lambda i: (0, 0)),
        ],
        out_specs=[
            pl.BlockSpec((RB, NCLS), lambda i: (i, 0)),
            pl.BlockSpec((RB, 1), lambda i: (i, 0)),
            pl.BlockSpec((RB, 1), lambda i: (i, 0)),
        ],
        out_shape=[
            jax.ShapeDtypeStruct((B, NCLS), jnp.float32),
            jax.ShapeDtypeStruct((B, 1), jnp.float32),
            jax.ShapeDtypeStruct((B, 1), jnp.float32),
        ],
    )(pooled_sums, tok_mat, mask, symb_feats, row0,
      W1a, W1b, b1.reshape(1, H), W2, b2.reshape(1, NCLS),
      Wsv, bsv.reshape(1, 1), Wcv, bcv.reshape(1, 1))
    return cls, sv.reshape(B), cv.reshape(B)


def kernel(tok_mat, mask, symb_feats, emb_table, W1, b1, W2, b2,
           Wsv, bsv, Wcv, bcv):
    V, EMB = emb_table.shape
    B, L = tok_mat.shape
    packed, vh = _tc_pack(emb_table.T)
    table_lin = packed.reshape(2 * vh, EMB)
    pooled_sums = _sc_pool(tok_mat.reshape(B * L), table_lin, vh, B, L)
    row0 = emb_table[0:1, :]
    W1a = W1[:EMB, :]
    W1b = W1[EMB:, :]
    return _tc_heads(pooled_sums, tok_mat, mask, symb_feats, row0,
                     W1a, W1b, b1, W2, b2, Wsv, bsv, Wcv, bcv)


# speedup vs baseline: 2.2132x; 1.1581x over previous
import functools

import jax
import jax.numpy as jnp
from jax import lax
from jax.experimental import pallas as pl
from jax.experimental.pallas import tpu as pltpu
from jax.experimental.pallas import tpu_sc as plsc

NC = 2
NS = 16
NW = NC * NS
LANES = 16


def _sc_pool(tok_flat, table_lin, vh, B, L):
    EMB = table_lin.shape[1]
    bags_per_w = B // NW
    G = 4
    GL = G * L
    n_groups = bags_per_w // G
    n_vec = EMB // LANES
    bag_chunks = [(0, 128), (128, L - 128)]

    mesh = plsc.VectorSubcoreMesh(core_axis_name="c", subcore_axis_name="s")

    @functools.partial(
        pl.kernel,
        out_type=jax.ShapeDtypeStruct((B, EMB), jnp.float32),
        mesh=mesh,
        scratch_types=[
            pltpu.VMEM((GL,), jnp.int32),
            pltpu.VMEM((GL,), jnp.int32),
            pltpu.VMEM((GL,), jnp.int32),
            pltpu.VMEM((GL,), jnp.int32),
            pltpu.VMEM((GL, EMB), jnp.float32),
            pltpu.VMEM((GL, EMB), jnp.float32),
            pltpu.VMEM((G, EMB), jnp.float32),
            pltpu.SemaphoreType.DMA,
            pltpu.SemaphoreType.DMA,
        ],
        compiler_params=pltpu.CompilerParams(use_tc_tiling_on_sc=False),
    )
    def pool(tok_hbm, table_hbm, out_hbm,
             idx0, idx1, jdx0, jdx1, rows0, rows1, acc_v, sem0, sem1):
        wid = lax.axis_index("s") * NC + lax.axis_index("c")
        base = wid * bags_per_w
        idx = (idx0, idx1)
        jdx = (jdx0, jdx1)
        rows = (rows0, rows1)
        sems = (sem0, sem1)

        def load_and_fire(g, pp):
            start = (base + g * G) * L
            pltpu.sync_copy(tok_hbm.at[pl.ds(start, GL)], idx[pp])
            for c in range(GL // LANES):
                v = idx[pp][pl.ds(c * LANES, LANES)]
                dbl = v + v
                jdx[pp][pl.ds(c * LANES, LANES)] = jnp.where(
                    v >= vh, dbl - (2 * vh - 1), dbl)
            for m in range(G):
                for off, sz in bag_chunks:
                    pltpu.async_copy(
                        table_hbm.at[jdx[pp].at[pl.ds(m * L + off, sz)]],
                        rows[pp].at[pl.ds(m * L + off, sz)],
                        sems[pp],
                    )

        def wait_and_acc(g, pp):
            for m in range(G):
                for off, sz in bag_chunks:
                    pltpu.make_async_copy(
                        table_hbm.at[jdx[pp].at[pl.ds(m * L + off, sz)]],
                        rows[pp].at[pl.ds(m * L + off, sz)],
                        sems[pp],
                    ).wait()
            for m in range(G):
                def row_body(j, accs):
                    return tuple(
                        accs[k] + rows[pp][j, pl.ds(k * LANES, LANES)]
                        for k in range(n_vec)
                    )
                zeros = tuple(jnp.zeros((LANES,), jnp.float32)
                              for _ in range(n_vec))
                accs = lax.fori_loop(m * L, (m + 1) * L, row_body, zeros)
                for k in range(n_vec):
                    acc_v[m, pl.ds(k * LANES, LANES)] = accs[k]
            pltpu.sync_copy(acc_v, out_hbm.at[pl.ds(base + g * G, G)])

        load_and_fire(0, 0)

        def body2(k, carry):
            g = 2 * k
            load_and_fire(g + 1, 1)
            wait_and_acc(g, 0)
            load_and_fire(g + 2, 0)
            wait_and_acc(g + 1, 1)
            return carry

        lax.fori_loop(0, n_groups // 2 - 1, body2, 0)
        g_last = n_groups - 2
        load_and_fire(g_last + 1, 1)
        wait_and_acc(g_last, 0)
        wait_and_acc(g_last + 1, 1)

    return pool(tok_flat, table_lin)


def _tc_pack(tableT):
    EMB, V = tableT.shape
    W = 4096
    nb = (V // 2 + W - 1) // W
    vh = nb * W

    def body(ta_ref, tb_ref, o_ref):
        o_ref[...] = jnp.concatenate(
            [jnp.transpose(ta_ref[...]), jnp.transpose(tb_ref[...])], axis=1)

    packed = pl.pallas_call(
        body,
        grid=(nb,),
        in_specs=[pl.BlockSpec((EMB, W), lambda i: (0, i)),
                  pl.BlockSpec((EMB, W),
                               lambda i: (0, jnp.minimum(i + nb,
                                                         V // W)))],
        out_specs=pl.BlockSpec((W, 2 * EMB), lambda i: (i, 0)),
        out_shape=jax.ShapeDtypeStruct((vh, 2 * EMB), jnp.float32),
    )(tableT, tableT)
    return packed, vh


def _tc_heads(pooled_sums, tok_mat, mask, symb_feats, row0,
              W1a, W1b, b1, W2, b2, Wsv, bsv, Wcv, bcv):
    B, L = tok_mat.shape
    EMB = pooled_sums.shape[1]
    S = symb_feats.shape[1]
    H = W1a.shape[1]
    NCLS = W2.shape[1]
    RB = 512
    grid = (B // RB,)

    def body(sums_ref, tok_ref, mask_ref, symb_ref, row0_ref,
             W1a_ref, W1b_ref, b1_ref, W2_ref, b2_ref,
             Wsv_ref, bsv_ref, Wcv_ref, bcv_ref,
             cls_ref, sv_ref, cv_ref):
        tok = tok_ref[...]
        cnt0 = jnp.sum((tok == 0).astype(jnp.float32), axis=1, keepdims=True)
        msum = jnp.sum(mask_ref[...], axis=1, keepdims=True)
        pooled = (sums_ref[...] - cnt0 * row0_ref[...]) / msum
        h = jnp.dot(pooled, W1a_ref[...], preferred_element_type=jnp.float32)
        h = h + jnp.dot(symb_ref[...], W1b_ref[...],
                        preferred_element_type=jnp.float32)
        h = jnp.maximum(h + b1_ref[...], 0.0)
        cls_ref[...] = (
            jnp.dot(h, W2_ref[...], preferred_element_type=jnp.float32)
            + b2_ref[...]
        )
        sv_ref[...] = (
            jnp.dot(pooled, Wsv_ref[...], preferred_element_type=jnp.float32)
            + bsv_ref[...]
        )
        cv_ref[...] = (
            jnp.dot(pooled, Wcv_ref[...], preferred_element_type=jnp.float32)
            + bcv_ref[...]
        )

    row_spec = pl.BlockSpec((RB, None), lambda i: (i, 0))
    full = pl.BlockSpec(lambda i: (0, 0))
    cls, sv, cv = pl.pallas_call(
        body,
        grid=grid,
        in_specs=[
            pl.BlockSpec((RB, EMB), lambda i: (i, 0)),
            pl.BlockSpec((RB, L), lambda i: (i, 0)),
            pl.BlockSpec((RB, L), lambda i: (i, 0)),
            pl.BlockSpec((RB, S), lambda i: (i, 0)),
            pl.BlockSpec((1, EMB), lambda i: (0, 0)),
            pl.BlockSpec((EMB, H), lambda i: (0, 0)),
            pl.BlockSpec((S, H), lambda i: (0, 0)),
            pl.BlockSpec((1, H), lambda i: (0, 0)),
            pl.BlockSpec((H, NCLS), lambda i: (0, 0)),
            pl.BlockSpec((1, NCLS), lambda i: (0, 0)),
            pl.BlockSpec((EMB, 1), lambda i: (0, 0)),
            pl.BlockSpec((1, 1), lambda i: (0, 0)),
            pl.BlockSpec((EMB, 1), lambda i: (0, 0)),
            pl.BlockSpec((1, 1), lambda i: (0, 0)),
        ],
        out_specs=[
            pl.BlockSpec((RB, NCLS), lambda i: (i, 0)),
            pl.BlockSpec((RB, 1), lambda i: (i, 0)),
            pl.BlockSpec((RB, 1), lambda i: (i, 0)),
        ],
        out_shape=[
            jax.ShapeDtypeStruct((B, NCLS), jnp.float32),
            jax.ShapeDtypeStruct((B, 1), jnp.float32),
            jax.ShapeDtypeStruct((B, 1), jnp.float32),
        ],
    )(pooled_sums, tok_mat, mask, symb_feats, row0,
      W1a, W1b, b1.reshape(1, H), W2, b2.reshape(1, NCLS),
      Wsv, bsv.reshape(1, 1), Wcv, bcv.reshape(1, 1))
    return cls, sv.reshape(B), cv.reshape(B)


def kernel(tok_mat, mask, symb_feats, emb_table, W1, b1, W2, b2,
           Wsv, bsv, Wcv, bcv):
    V, EMB = emb_table.shape
    B, L = tok_mat.shape
    packed, vh = _tc_pack(emb_table.T)
    table_lin = packed.reshape(2 * vh, EMB)
    pooled_sums = _sc_pool(tok_mat.reshape(B * L), table_lin, vh, B, L)
    row0 = emb_table[0:1, :]
    W1a = W1[:EMB, :]
    W1b = W1[EMB:, :]
    return _tc_heads(pooled_sums, tok_mat, mask, symb_feats, row0,
                     W1a, W1b, b1, W2, b2, Wsv, bsv, Wcv, bcv)


# speedup vs baseline: 2.3985x; 1.0837x over previous
import functools

import jax
import jax.numpy as jnp
from jax import lax
from jax.experimental import pallas as pl
from jax.experimental.pallas import tpu as pltpu
from jax.experimental.pallas import tpu_sc as plsc

NC = 2
NS = 16
NW = NC * NS
LANES = 16


def _sc_pool(tok_flat, table_lin, vh, B, L):
    EMB = table_lin.shape[1]
    bags_per_w = B // NW
    G = 4
    GL = G * L
    n_groups = bags_per_w // G
    n_vec = EMB // LANES
    bag_chunks = [(0, 128), (128, L - 128)]

    mesh = plsc.VectorSubcoreMesh(core_axis_name="c", subcore_axis_name="s")

    @functools.partial(
        pl.kernel,
        out_type=jax.ShapeDtypeStruct((B, EMB), jnp.float32),
        mesh=mesh,
        scratch_types=[
            pltpu.VMEM((GL,), jnp.int32),
            pltpu.VMEM((GL,), jnp.int32),
            pltpu.VMEM((GL,), jnp.int32),
            pltpu.VMEM((GL,), jnp.int32),
            pltpu.VMEM((GL, EMB), jnp.float32),
            pltpu.VMEM((GL, EMB), jnp.float32),
            pltpu.VMEM((G, EMB), jnp.float32),
            pltpu.SemaphoreType.DMA,
            pltpu.SemaphoreType.DMA,
        ],
        compiler_params=pltpu.CompilerParams(use_tc_tiling_on_sc=False),
    )
    def pool(tok_hbm, table_hbm, out_hbm,
             idx0, idx1, jdx0, jdx1, rows0, rows1, acc_v, sem0, sem1):
        wid = lax.axis_index("s") * NC + lax.axis_index("c")
        base = wid * bags_per_w
        idx = (idx0, idx1)
        jdx = (jdx0, jdx1)
        rows = (rows0, rows1)
        sems = (sem0, sem1)

        def load_and_fire(g, pp):
            start = (base + g * G) * L
            pltpu.sync_copy(tok_hbm.at[pl.ds(start, GL)], idx[pp])
            for c in range(GL // LANES):
                v = idx[pp][pl.ds(c * LANES, LANES)]
                dbl = v + v
                jdx[pp][pl.ds(c * LANES, LANES)] = jnp.where(
                    v >= vh, dbl - (2 * vh - 1), dbl)
            for m in range(G):
                for off, sz in bag_chunks:
                    pltpu.async_copy(
                        table_hbm.at[jdx[pp].at[pl.ds(m * L + off, sz)]],
                        rows[pp].at[pl.ds(m * L + off, sz)],
                        sems[pp],
                    )

        def wait_and_acc(g, pp):
            for m in range(G):
                for off, sz in bag_chunks:
                    pltpu.make_async_copy(
                        table_hbm.at[jdx[pp].at[pl.ds(m * L + off, sz)]],
                        rows[pp].at[pl.ds(m * L + off, sz)],
                        sems[pp],
                    ).wait()
            for m in range(G):
                def row_body(j, accs):
                    return tuple(
                        accs[k] + rows[pp][j, pl.ds(k * LANES, LANES)]
                        for k in range(n_vec)
                    )
                zeros = tuple(jnp.zeros((LANES,), jnp.float32)
                              for _ in range(n_vec))
                accs = lax.fori_loop(m * L, (m + 1) * L, row_body, zeros)
                for k in range(n_vec):
                    acc_v[m, pl.ds(k * LANES, LANES)] = accs[k]
            pltpu.sync_copy(acc_v, out_hbm.at[pl.ds(base + g * G, G)])

        load_and_fire(0, 0)

        def body2(k, carry):
            g = 2 * k
            load_and_fire(g + 1, 1)
            wait_and_acc(g, 0)
            load_and_fire(g + 2, 0)
            wait_and_acc(g + 1, 1)
            return carry

        lax.fori_loop(0, n_groups // 2 - 1, body2, 0)
        g_last = n_groups - 2
        load_and_fire(g_last + 1, 1)
        wait_and_acc(g_last, 0)
        wait_and_acc(g_last + 1, 1)

    return pool(tok_flat, table_lin)


def _tc_pack(tableT):
    EMB, V = tableT.shape
    W = 8192
    nb = (V // 2 + W - 1) // W
    vh = nb * W

    def body(ta_ref, tb_ref, o_ref):
        o_ref[...] = jnp.concatenate(
            [jnp.transpose(ta_ref[...]), jnp.transpose(tb_ref[...])], axis=1)

    packed = pl.pallas_call(
        body,
        grid=(nb,),
        in_specs=[pl.BlockSpec((EMB, W), lambda i: (0, i)),
                  pl.BlockSpec((EMB, W),
                               lambda i: (0, jnp.minimum(i + nb,
                                                         V // W)))],
        out_specs=pl.BlockSpec((W, 2 * EMB), lambda i: (i, 0)),
        out_shape=jax.ShapeDtypeStruct((vh, 2 * EMB), jnp.float32),
    )(tableT, tableT)
    return packed, vh


def _tc_heads(pooled_sums, tok_mat, mask, symb_feats, row0,
              W1a, W1b, b1, W2, b2, Wsv, bsv, Wcv, bcv):
    B, L = tok_mat.shape
    EMB = pooled_sums.shape[1]
    S = symb_feats.shape[1]
    H = W1a.shape[1]
    NCLS = W2.shape[1]
    RB = 512
    grid = (B // RB,)

    def body(sums_ref, tok_ref, mask_ref, symb_ref, row0_ref,
             W1a_ref, W1b_ref, b1_ref, W2_ref, b2_ref,
             Wsv_ref, bsv_ref, Wcv_ref, bcv_ref,
             cls_ref, sv_ref, cv_ref):
        tok = tok_ref[...]
        cnt0 = jnp.sum((tok == 0).astype(jnp.float32), axis=1, keepdims=True)
        msum = jnp.sum(mask_ref[...], axis=1, keepdims=True)
        pooled = (sums_ref[...] - cnt0 * row0_ref[...]) / msum
        h = jnp.dot(pooled, W1a_ref[...], preferred_element_type=jnp.float32)
        h = h + jnp.dot(symb_ref[...], W1b_ref[...],
                        preferred_element_type=jnp.float32)
        h = jnp.maximum(h + b1_ref[...], 0.0)
        cls_ref[...] = (
            jnp.dot(h, W2_ref[...], preferred_element_type=jnp.float32)
            + b2_ref[...]
        )
        sv_ref[...] = (
            jnp.dot(pooled, Wsv_ref[...], preferred_element_type=jnp.float32)
            + bsv_ref[...]
        )
        cv_ref[...] = (
            jnp.dot(pooled, Wcv_ref[...], preferred_element_type=jnp.float32)
            + bcv_ref[...]
        )

    row_spec = pl.BlockSpec((RB, None), lambda i: (i, 0))
    full = pl.BlockSpec(lambda i: (0, 0))
    cls, sv, cv = pl.pallas_call(
        body,
        grid=grid,
        in_specs=[
            pl.BlockSpec((RB, EMB), lambda i: (i, 0)),
            pl.BlockSpec((RB, L), lambda i: (i, 0)),
            pl.BlockSpec((RB, L), lambda i: (i, 0)),
            pl.BlockSpec((RB, S), lambda i: (i, 0)),
            pl.BlockSpec((1, EMB), lambda i: (0, 0)),
            pl.BlockSpec((EMB, H), lambda i: (0, 0)),
            pl.BlockSpec((S, H), lambda i: (0, 0)),
            pl.BlockSpec((1, H), lambda i: (0, 0)),
            pl.BlockSpec((H, NCLS), lambda i: (0, 0)),
            pl.BlockSpec((1, NCLS), lambda i: (0, 0)),
            pl.BlockSpec((EMB, 1), lambda i: (0, 0)),
            pl.BlockSpec((1, 1), lambda i: (0, 0)),
            pl.BlockSpec((EMB, 1), lambda i: (0, 0)),
            pl.BlockSpec((1, 1), lambda i: (0, 0)),
        ],
        out_specs=[
            pl.BlockSpec((RB, NCLS), lambda i: (i, 0)),
            pl.BlockSpec((RB, 1), lambda i: (i, 0)),
            pl.BlockSpec((RB, 1), lambda i: (i, 0)),
        ],
        out_shape=[
            jax.ShapeDtypeStruct((B, NCLS), jnp.float32),
            jax.ShapeDtypeStruct((B, 1), jnp.float32),
            jax.ShapeDtypeStruct((B, 1), jnp.float32),
        ],
    )(pooled_sums, tok_mat, mask, symb_feats, row0,
      W1a, W1b, b1.reshape(1, H), W2, b2.reshape(1, NCLS),
      Wsv, bsv.reshape(1, 1), Wcv, bcv.reshape(1, 1))
    return cls, sv.reshape(B), cv.reshape(B)


def kernel(tok_mat, mask, symb_feats, emb_table, W1, b1, W2, b2,
           Wsv, bsv, Wcv, bcv):
    V, EMB = emb_table.shape
    B, L = tok_mat.shape
    packed, vh = _tc_pack(emb_table.T)
    table_lin = packed.reshape(2 * vh, EMB)
    pooled_sums = _sc_pool(tok_mat.reshape(B * L), table_lin, vh, B, L)
    row0 = emb_table[0:1, :]
    W1a = W1[:EMB, :]
    W1b = W1[EMB:, :]
    return _tc_heads(pooled_sums, tok_mat, mask, symb_feats, row0,
                     W1a, W1b, b1, W2, b2, Wsv, bsv, Wcv, bcv)
